# baseline (speedup 1.0000x reference)
import functools

import jax
import jax.numpy as jnp
from jax import lax
from jax.experimental import pallas as pl
from jax.experimental.pallas import tpu as pltpu
from jax.experimental.pallas import tpu_sc as plsc

N_NODES = 10000
N_EDGES = 320000
F = 128

NC = 2
NS = 16
NW = NC * NS
C = 128
K = 80
E_PAD = NW * K * C
K0 = 80
K1 = 2 * K - K0
NROW = 10240
RPT = NROW // NS
ZCH = RPT // C
R_BLK = 512
GRID = NROW // R_BLK

_MESH = plsc.VectorSubcoreMesh(core_axis_name="c", subcore_axis_name="s")


@functools.partial(
    pl.kernel,
    out_type=jax.ShapeDtypeStruct((NC, NROW, F), jnp.float32),
    mesh=_MESH,
    scratch_types=[
        pltpu.VMEM_SHARED((NROW, F), jnp.float32),
        pltpu.VMEM((C,), jnp.int32),
        pltpu.VMEM((C,), jnp.int32),
        pltpu.VMEM((C, F), jnp.float32),
        pltpu.SemaphoreType.DMA,
    ])
def _sc_agg(table, src, dst, z128, p_out, acc, si, di, rows0, gsem):
  c = lax.axis_index("c")
  s = lax.axis_index("s")
  wid = c * NS + s
  base = s * RPT
  pltpu.sync_copy(z128, rows0)
  for k in range(ZCH):
    pltpu.sync_copy(rows0, acc.at[pl.ds(base + k * C, C)])
  plsc.subcore_barrier()

  def step(j, carry):
    pltpu.sync_copy(src.at[wid, j], si)
    pltpu.sync_copy(dst.at[wid, j], di)
    cp = pltpu.make_async_copy(table.at[si], rows0, gsem)
    cp.start()
    cp.wait()
    pltpu.sync_copy(rows0, acc.at[di], add=True)
    return carry

  kc = jnp.where(c == 0, K0, K1)
  lax.fori_loop(0, kc, step, 0)
  plsc.subcore_barrier()
  for k in range(ZCH):
    r = base + k * C
    pltpu.sync_copy(acc.at[pl.ds(r, C)], rows0)
    pltpu.sync_copy(rows0, p_out.at[c, pl.ds(r, C)])


@functools.partial(
    pl.kernel,
    out_type=jax.ShapeDtypeStruct((NC, NROW, F), jnp.float32),
    mesh=_MESH,
    scratch_types=[
        pltpu.VMEM_SHARED((NROW, F), jnp.float32),
        pltpu.VMEM((C,), jnp.int32),
        pltpu.VMEM((C, F), jnp.float32),
        pltpu.SemaphoreType.DMA,
    ])
def _sc_deg(dst, z128, o128, pd_out, acc, di, ones_v, sem):
  c = lax.axis_index("c")
  s = lax.axis_index("s")
  wid = c * NS + s
  base = s * RPT
  pltpu.sync_copy(z128, ones_v)
  for k in range(ZCH):
    pltpu.sync_copy(ones_v, acc.at[pl.ds(base + k * C, C)])
  pltpu.sync_copy(o128, ones_v)
  plsc.subcore_barrier()

  def step(j, carry):
    pltpu.sync_copy(dst.at[wid, j], di)
    pltpu.sync_copy(ones_v, acc.at[di], add=True)
    return carry

  kc = jnp.where(c == 0, K0, K1)
  lax.fori_loop(0, kc, step, 0)
  plsc.subcore_barrier()
  for k in range(ZCH):
    r = base + k * C
    pltpu.sync_copy(acc.at[pl.ds(r, C)], ones_v)
    pltpu.sync_copy(ones_v, pd_out.at[c, pl.ds(r, C)])


def _make_dense(first, relu):
  def body(*refs):
    if first:
      h_ref, p_ref, d_ref, ws_ref, wn_ref, b_ref, o_ref, do_ref = refs
      deg = jnp.maximum(d_ref[0] + d_ref[1], 1.0)
      do_ref[...] = deg
    else:
      h_ref, p_ref, d_ref, ws_ref, wn_ref, b_ref, o_ref = refs
      deg = d_ref[...]
    agg = p_ref[0] + p_ref[1]
    hn = agg / deg
    out = (jnp.dot(h_ref[...], ws_ref[...], preferred_element_type=jnp.float32)
           + jnp.dot(hn, wn_ref[...], preferred_element_type=jnp.float32)
           + b_ref[...])
    if relu:
      out = jnp.maximum(out, 0.0)
    o_ref[...] = out

  in_specs = [
      pl.BlockSpec((R_BLK, F), lambda i: (i, 0)),
      pl.BlockSpec((NC, R_BLK, F), lambda i: (0, i, 0)),
      (pl.BlockSpec((NC, R_BLK, F), lambda i: (0, i, 0)) if first
       else pl.BlockSpec((R_BLK, F), lambda i: (i, 0))),
      pl.BlockSpec((F, F), lambda i: (0, 0)),
      pl.BlockSpec((F, F), lambda i: (0, 0)),
      pl.BlockSpec((1, F), lambda i: (0, 0)),
  ]
  out_shape = [jax.ShapeDtypeStruct((NROW, F), jnp.float32)]
  out_specs = [pl.BlockSpec((R_BLK, F), lambda i: (i, 0))]
  if first:
    out_shape.append(jax.ShapeDtypeStruct((NROW, F), jnp.float32))
    out_specs.append(pl.BlockSpec((R_BLK, F), lambda i: (i, 0)))

  return pl.pallas_call(
      body,
      grid=(GRID,),
      in_specs=in_specs,
      out_specs=out_specs,
      out_shape=out_shape,
  )


_dense_first = _make_dense(True, True)
_dense_mid = _make_dense(False, True)
_dense_last = _make_dense(False, False)


def _pack_idx(flat, pad_value):
  km = max(K0, K1)
  full = jnp.concatenate(
      [flat, jnp.full((E_PAD - N_EDGES,), pad_value, jnp.int32)])
  e0 = full[:NS * K0 * C].reshape(NS, K0, C)
  e1 = full[NS * K0 * C:].reshape(NS, K1, C)
  pad = lambda e, k: jnp.concatenate(
      [e, jnp.full((NS, km - k, C), pad_value, jnp.int32)], axis=1)
  return jnp.concatenate([pad(e0, K0), pad(e1, K1)], axis=0)


@jax.jit
def kernel(x, edge_index, W_self1, W_neigh1, b1, W_self2, W_neigh2, b2,
           W_self3, W_neigh3, b3):
  src = _pack_idx(edge_index[0], 0)
  dst = _pack_idx(edge_index[1], N_NODES)
  h = jnp.concatenate([x, jnp.zeros((NROW - N_NODES, F), jnp.float32)])
  z128 = jnp.zeros((C, F), jnp.float32)
  o128 = jnp.ones((C, F), jnp.float32)

  pd = _sc_deg(dst, z128, o128)
  p = _sc_agg(h, src, dst, z128)
  h, deg = _dense_first(h, p, pd, W_self1.T, W_neigh1.T, b1[None, :])
  p = _sc_agg(h, src, dst, z128)
  (h,) = _dense_mid(h, p, deg, W_self2.T, W_neigh2.T, b2[None, :])
  p = _sc_agg(h, src, dst, z128)
  (out,) = _dense_last(h, p, deg, W_self3.T, W_neigh3.T, b3[None, :])
  return out[:N_NODES]


# speedup vs baseline: 1.0609x; 1.0609x over previous
import functools

import jax
import jax.numpy as jnp
from jax import lax
from jax.experimental import pallas as pl
from jax.experimental.pallas import tpu as pltpu
from jax.experimental.pallas import tpu_sc as plsc

N_NODES = 10000
N_EDGES = 320000
F = 128

NC = 2
NS = 16
NW = NC * NS
C = 128
K = 80
E_PAD = NW * K * C

NROW = 10240
RPT = NROW // NS
ZCH = RPT // C
R_BLK = 512
GRID = NROW // R_BLK

_MESH = plsc.VectorSubcoreMesh(core_axis_name="c", subcore_axis_name="s")


@functools.partial(
    pl.kernel,
    out_type=jax.ShapeDtypeStruct((NC, NROW, F), jnp.float32),
    mesh=_MESH,
    scratch_types=[
        pltpu.VMEM_SHARED((NROW, F), jnp.float32),
        pltpu.VMEM((C,), jnp.int32),
        pltpu.VMEM((C,), jnp.int32),
        pltpu.VMEM((C, F), jnp.float32),
        pltpu.SemaphoreType.DMA,
    ])
def _sc_agg(table, src, dst, z128, p_out, acc, si, di, rows0, gsem):
  c = lax.axis_index("c")
  s = lax.axis_index("s")
  wid = c * NS + s
  base = s * RPT
  pltpu.sync_copy(z128, rows0)
  for k in range(ZCH):
    pltpu.sync_copy(rows0, acc.at[pl.ds(base + k * C, C)])
  plsc.subcore_barrier()

  def step(j, carry):
    pltpu.sync_copy(src.at[wid, j], si)
    pltpu.sync_copy(dst.at[wid, j], di)
    cp = pltpu.make_async_copy(table.at[si], rows0, gsem)
    cp.start()
    cp.wait()
    pltpu.sync_copy(rows0, acc.at[di], add=True)
    return carry

  lax.fori_loop(0, K, step, 0)
  plsc.subcore_barrier()
  for k in range(ZCH):
    r = base + k * C
    pltpu.sync_copy(acc.at[pl.ds(r, C)], rows0)
    pltpu.sync_copy(rows0, p_out.at[c, pl.ds(r, C)])


@functools.partial(
    pl.kernel,
    out_type=jax.ShapeDtypeStruct((NC, NROW, F), jnp.float32),
    mesh=_MESH,
    scratch_types=[
        pltpu.VMEM_SHARED((NROW, F), jnp.float32),
        pltpu.VMEM((C,), jnp.int32),
        pltpu.VMEM((C, F), jnp.float32),
        pltpu.SemaphoreType.DMA,
    ])
def _sc_deg(dst, z128, o128, pd_out, acc, di, ones_v, sem):
  c = lax.axis_index("c")
  s = lax.axis_index("s")
  wid = c * NS + s
  base = s * RPT
  pltpu.sync_copy(z128, ones_v)
  for k in range(ZCH):
    pltpu.sync_copy(ones_v, acc.at[pl.ds(base + k * C, C)])
  pltpu.sync_copy(o128, ones_v)
  plsc.subcore_barrier()

  def step(j, carry):
    pltpu.sync_copy(dst.at[wid, j], di)
    pltpu.sync_copy(ones_v, acc.at[di], add=True)
    return carry

  lax.fori_loop(0, K, step, 0)
  plsc.subcore_barrier()
  for k in range(ZCH):
    r = base + k * C
    pltpu.sync_copy(acc.at[pl.ds(r, C)], ones_v)
    pltpu.sync_copy(ones_v, pd_out.at[c, pl.ds(r, C)])


def _make_dense(first, relu):
  def body(*refs):
    if first:
      h_ref, p_ref, d_ref, ws_ref, wn_ref, b_ref, o_ref, do_ref = refs
      deg = jnp.maximum(d_ref[0] + d_ref[1], 1.0)
      do_ref[...] = deg
    else:
      h_ref, p_ref, d_ref, ws_ref, wn_ref, b_ref, o_ref = refs
      deg = d_ref[...]
    agg = p_ref[0] + p_ref[1]
    hn = agg / deg
    out = (jnp.dot(h_ref[...], ws_ref[...], preferred_element_type=jnp.float32)
           + jnp.dot(hn, wn_ref[...], preferred_element_type=jnp.float32)
           + b_ref[...])
    if relu:
      out = jnp.maximum(out, 0.0)
    o_ref[...] = out

  in_specs = [
      pl.BlockSpec((R_BLK, F), lambda i: (i, 0)),
      pl.BlockSpec((NC, R_BLK, F), lambda i: (0, i, 0)),
      (pl.BlockSpec((NC, R_BLK, F), lambda i: (0, i, 0)) if first
       else pl.BlockSpec((R_BLK, F), lambda i: (i, 0))),
      pl.BlockSpec((F, F), lambda i: (0, 0)),
      pl.BlockSpec((F, F), lambda i: (0, 0)),
      pl.BlockSpec((1, F), lambda i: (0, 0)),
  ]
  out_shape = [jax.ShapeDtypeStruct((NROW, F), jnp.float32)]
  out_specs = [pl.BlockSpec((R_BLK, F), lambda i: (i, 0))]
  if first:
    out_shape.append(jax.ShapeDtypeStruct((NROW, F), jnp.float32))
    out_specs.append(pl.BlockSpec((R_BLK, F), lambda i: (i, 0)))

  return pl.pallas_call(
      body,
      grid=(GRID,),
      in_specs=in_specs,
      out_specs=out_specs,
      out_shape=out_shape,
  )


_dense_first = _make_dense(True, True)
_dense_mid = _make_dense(False, True)
_dense_last = _make_dense(False, False)


def _pack_idx(flat, pad_value):
  return jnp.concatenate(
      [flat, jnp.full((E_PAD - N_EDGES,), pad_value, jnp.int32)]
  ).reshape(NW, K, C)


@jax.jit
def kernel(x, edge_index, W_self1, W_neigh1, b1, W_self2, W_neigh2, b2,
           W_self3, W_neigh3, b3):
  src = _pack_idx(edge_index[0], 0)
  dst = _pack_idx(edge_index[1], N_NODES)
  h = jnp.concatenate([x, jnp.zeros((NROW - N_NODES, F), jnp.float32)])
  z128 = jnp.zeros((C, F), jnp.float32)
  o128 = jnp.ones((C, F), jnp.float32)

  pd = _sc_deg(dst, z128, o128)
  p = _sc_agg(h, src, dst, z128)
  h, deg = _dense_first(h, p, pd, W_self1.T, W_neigh1.T, b1[None, :])
  p = _sc_agg(h, src, dst, z128)
  (h,) = _dense_mid(h, p, deg, W_self2.T, W_neigh2.T, b2[None, :])
  p = _sc_agg(h, src, dst, z128)
  (out,) = _dense_last(h, p, deg, W_self3.T, W_neigh3.T, b3[None, :])
  return out[:N_NODES]


# speedup vs baseline: 1.4653x; 1.3812x over previous
import functools

import jax
import jax.numpy as jnp
from jax import lax
from jax.experimental import pallas as pl
from jax.experimental.pallas import tpu as pltpu
from jax.experimental.pallas import tpu_sc as plsc

N_NODES = 10000
N_EDGES = 320000
F = 128

NC = 2
NS = 16
NW = NC * NS
C = 128
K = 79
E_PAD = NW * K * C

NROW = 10240
RPT = NROW // NS
ZCH = RPT // C
R_BLK = 512
GRID = NROW // R_BLK

_MESH = plsc.VectorSubcoreMesh(core_axis_name="c", subcore_axis_name="s")


@functools.partial(
    pl.kernel,
    out_type=jax.ShapeDtypeStruct((NC, NROW, F), jnp.float32),
    mesh=_MESH,
    scratch_types=[
        pltpu.VMEM_SHARED((NROW, F), jnp.float32),
        pltpu.VMEM((C,), jnp.int32),
        pltpu.VMEM((C,), jnp.int32),
        pltpu.VMEM((C, F), jnp.float32),
        pltpu.SemaphoreType.DMA,
    ])
def _sc_agg(table, src, dst, z128, p_out, acc, si, di, rows0, gsem):
  c = lax.axis_index("c")
  s = lax.axis_index("s")
  wid = c * NS + s
  base = s * RPT
  pltpu.sync_copy(z128, rows0)
  for k in range(ZCH):
    pltpu.sync_copy(rows0, acc.at[pl.ds(base + k * C, C)])
  plsc.subcore_barrier()

  def step(j, carry):
    pltpu.sync_copy(src.at[wid, j], si)
    pltpu.sync_copy(dst.at[wid, j], di)
    cp = pltpu.make_async_copy(table.at[si], rows0, gsem)
    cp.start()
    cp.wait()
    pltpu.sync_copy(rows0, acc.at[di], add=True)
    return carry

  lax.fori_loop(0, K, step, 0)
  plsc.subcore_barrier()
  for k in range(ZCH):
    r = base + k * C
    pltpu.sync_copy(acc.at[pl.ds(r, C)], rows0)
    pltpu.sync_copy(rows0, p_out.at[c, pl.ds(r, C)])


@functools.partial(
    pl.kernel,
    out_type=jax.ShapeDtypeStruct((NC, NROW, F), jnp.float32),
    mesh=_MESH,
    scratch_types=[
        pltpu.VMEM_SHARED((NROW, F), jnp.float32),
        pltpu.VMEM((C,), jnp.int32),
        pltpu.VMEM((C, F), jnp.float32),
        pltpu.SemaphoreType.DMA,
    ])
def _sc_deg(dst, z128, o128, pd_out, acc, di, ones_v, sem):
  c = lax.axis_index("c")
  s = lax.axis_index("s")
  wid = c * NS + s
  base = s * RPT
  pltpu.sync_copy(z128, ones_v)
  for k in range(ZCH):
    pltpu.sync_copy(ones_v, acc.at[pl.ds(base + k * C, C)])
  pltpu.sync_copy(o128, ones_v)
  plsc.subcore_barrier()

  def step(j, carry):
    pltpu.sync_copy(dst.at[wid, j], di)
    pltpu.sync_copy(ones_v, acc.at[di], add=True)
    return carry

  lax.fori_loop(0, K, step, 0)
  plsc.subcore_barrier()
  for k in range(ZCH):
    r = base + k * C
    pltpu.sync_copy(acc.at[pl.ds(r, C)], ones_v)
    pltpu.sync_copy(ones_v, pd_out.at[c, pl.ds(r, C)])


def _make_dense(first, relu):
  def body(*refs):
    if first:
      h_ref, p_ref, d_ref, ws_ref, wn_ref, b_ref, o_ref, do_ref = refs
      deg = jnp.maximum(d_ref[0] + d_ref[1], 1.0)
      do_ref[...] = deg
    else:
      h_ref, p_ref, d_ref, ws_ref, wn_ref, b_ref, o_ref = refs
      deg = d_ref[...]
    agg = p_ref[0] + p_ref[1]
    hn = agg / deg
    out = (jnp.dot(h_ref[...], ws_ref[...], preferred_element_type=jnp.float32)
           + jnp.dot(hn, wn_ref[...], preferred_element_type=jnp.float32)
           + b_ref[...])
    if relu:
      out = jnp.maximum(out, 0.0)
    o_ref[...] = out

  in_specs = [
      pl.BlockSpec((R_BLK, F), lambda i: (i, 0)),
      pl.BlockSpec((NC, R_BLK, F), lambda i: (0, i, 0)),
      (pl.BlockSpec((NC, R_BLK, F), lambda i: (0, i, 0)) if first
       else pl.BlockSpec((R_BLK, F), lambda i: (i, 0))),
      pl.BlockSpec((F, F), lambda i: (0, 0)),
      pl.BlockSpec((F, F), lambda i: (0, 0)),
      pl.BlockSpec((1, F), lambda i: (0, 0)),
  ]
  out_shape = [jax.ShapeDtypeStruct((NROW, F), jnp.float32)]
  out_specs = [pl.BlockSpec((R_BLK, F), lambda i: (i, 0))]
  if first:
    out_shape.append(jax.ShapeDtypeStruct((NROW, F), jnp.float32))
    out_specs.append(pl.BlockSpec((R_BLK, F), lambda i: (i, 0)))

  return pl.pallas_call(
      body,
      grid=(GRID,),
      in_specs=in_specs,
      out_specs=out_specs,
      out_shape=out_shape,
  )


_dense_first = _make_dense(True, True)
_dense_mid = _make_dense(False, True)
_dense_last = _make_dense(False, False)


def _pack_idx(flat, pad_value):
  return jnp.concatenate(
      [flat, jnp.full((E_PAD - N_EDGES,), pad_value, jnp.int32)]
  ).reshape(NW, K, C)


@jax.jit
def kernel(x, edge_index, W_self1, W_neigh1, b1, W_self2, W_neigh2, b2,
           W_self3, W_neigh3, b3):
  src = _pack_idx(edge_index[0], 0)
  dst = _pack_idx(edge_index[1], N_NODES)
  h = jnp.concatenate([x, jnp.zeros((NROW - N_NODES, F), jnp.float32)])
  z128 = jnp.zeros((C, F), jnp.float32)
  o128 = jnp.ones((C, F), jnp.float32)

  pd = _sc_deg(dst, z128, o128)
  p = _sc_agg(h, src, dst, z128)
  h, deg = _dense_first(h, p, pd, W_self1.T, W_neigh1.T, b1[None, :])
  p = _sc_agg(h, src, dst, z128)
  (h,) = _dense_mid(h, p, deg, W_self2.T, W_neigh2.T, b2[None, :])
  p = _sc_agg(h, src, dst, z128)
  (out,) = _dense_last(h, p, deg, W_self3.T, W_neigh3.T, b3[None, :])
  return out[:N_NODES]
